# baseline (speedup 1.0000x reference)
import functools

import numpy as np
import jax
import jax.numpy as jnp
from jax import lax
from jax.experimental import pallas as pl
from jax.experimental.pallas import tpu as pltpu
from jax.experimental.pallas import tpu_sc as plsc

L = 24
NC, NS, LN = 2, 16, 16
NW = NC * NS
WMAX = L - 1


def _build_static():
    widths = []
    for level in range(1, L):
        for _pos in range(L - level):
            widths.append(level)
    widths = np.asarray(widths, np.int64)
    n_seg = len(widths)
    offs = np.concatenate([[0], np.cumsum(widths)[:-1]])

    perm, seg_id, mdum = [], [], []
    ppos = {}
    for p in range(int(widths.max())):
        for s in np.nonzero(widths > p)[0]:
            ppos[(s, p)] = len(perm)
            perm.append(offs[s] + p)
            seg_id.append(s)
            mdum.append(1.0)
        while len(perm) % LN:
            t = len(perm)
            perm.append(0)
            seg_id.append(n_seg + (t % LN))
            mdum.append(0.0)
    t2 = len(perm)
    sid = np.asarray(seg_id, np.int32)
    for c in range(t2 // LN):
        assert len(set(sid[c * LN:(c + 1) * LN].tolist())) == LN

    ntbl = (n_seg + LN) * WMAX
    ntbl += (LN - ntbl % LN) % LN
    tbl = np.zeros((ntbl,), np.int32)
    for (s, p), v in ppos.items():
        tbl[s * WMAX + p] = v
    for k in range(LN):
        tbl[(n_seg + k) * WMAX] = t2 + k

    pi, pj = [], []
    for s in range(n_seg):
        for i in range(int(widths[s])):
            for j in range(i + 1, int(widths[s])):
                pi.append(ppos[(s, i)])
                pj.append(ppos[(s, j)])
    while len(pi) % LN:
        pi.append(0)
        pj.append(0)
    pij = (np.asarray(pi, np.int64)
           | (np.asarray(pj, np.int64) << 16)).astype(np.int32)
    nbin = -(-(n_seg + LN) // LN) * LN
    return dict(
        t2=t2, nbin=nbin, perm=np.asarray(perm, np.int64),
        sid=sid, md=np.asarray(mdum, np.float32), tbl=tbl, pij=pij)


_ST = _build_static()
T2 = _ST["t2"]
NPV = len(_ST["pij"]) // LN
NBIN = _ST["nbin"]
NCHUNK = T2 // LN
NTBL = len(_ST["tbl"])
RRN = T2 + LN
SLAB = 4 * T2 + 64


def _sc_loss(slab, w, sid, tbl, pij):
    b = slab.shape[0]
    rows_per = b // NW
    mesh = plsc.VectorSubcoreMesh(core_axis_name="c", subcore_axis_name="s")
    f32 = jnp.float32
    i32 = jnp.int32
    c100 = f32(100.0)

    @functools.partial(
        pl.kernel,
        out_type=jax.ShapeDtypeStruct((NW, LN), f32),
        mesh=mesh,
        compiler_params=pltpu.CompilerParams(needs_layout_passes=False),
        scratch_types=[
            pltpu.VMEM((w.shape[0],), f32),
            pltpu.VMEM((T2,), i32),
            pltpu.VMEM((NTBL,), i32),
            pltpu.VMEM((len(_ST["pij"]),), i32),
            pltpu.VMEM((rows_per * SLAB,), f32),
            pltpu.VMEM((T2,), f32),
            pltpu.VMEM((T2,), f32),
            pltpu.VMEM((RRN,), f32),
            pltpu.VMEM((RRN,), f32),
            pltpu.VMEM((T2,), f32),
            pltpu.VMEM((T2,), f32),
            pltpu.VMEM((T2,), f32),
            pltpu.VMEM((NBIN,), f32),
            pltpu.VMEM((NBIN,), f32),
            pltpu.VMEM((LN,), f32),
        ] + [pltpu.SemaphoreType.DMA] * rows_per,
    )
    def k(slab_h, w_h, sid_h, tbl_h, pij_h, out_h,
          w_v, sid_v, tbl_v, pij_v, rows_v, ea_v, er_v, rr_v, dr_v,
          r_v, a_v, d_v, dena_v, denr_v, acc_v, *sems):
        wid = lax.axis_index("s") * NC + lax.axis_index("c")
        base = wid * rows_per
        copies = [
            pltpu.async_copy(slab_h.at[base + r],
                             rows_v.at[pl.ds(r * SLAB, SLAB)], sems[r])
            for r in range(rows_per)
        ]
        pltpu.sync_copy(w_h, w_v)
        pltpu.sync_copy(sid_h, sid_v)
        pltpu.sync_copy(tbl_h, tbl_v)
        pltpu.sync_copy(pij_h, pij_v)
        zero = jnp.zeros((LN,), f32)

        def zero_big(i):
            rr_v[pl.ds(i * LN, LN)] = zero
            dr_v[pl.ds(i * LN, LN)] = zero
        plsc.parallel_loop(0, RRN // LN, unroll=4)(zero_big)

        acc = zero
        for r in range(rows_per):
            copies[r].wait()
            roff = r * SLAB

            def zero_bins(i):
                dena_v[pl.ds(i * LN, LN)] = zero
                denr_v[pl.ds(i * LN, LN)] = zero
            plsc.parallel_loop(0, NBIN // LN, unroll=4)(zero_bins)

            def p1(i):
                s = pl.ds(i * LN, LN)
                ea = jnp.exp(rows_v[pl.ds(roff + i * LN, LN)])
                ea_v[s] = ea
                sb = sid_v[s]
                plsc.addupdate_scatter(dena_v, [sb], ea)
                me = rows_v[pl.ds(roff + T2 + i * LN, LN)]
                ru = plsc.bitcast(rows_v[pl.ds(roff + 2 * T2 + i * LN, LN)], i32)
                ix = plsc.bitcast(rows_v[pl.ds(roff + 3 * T2 + i * LN, LN)], i32)
                g = plsc.load_gather(w_v, [ru]) * me
                tgt = plsc.load_gather(tbl_v, [sb * WMAX + ix])
                plsc.addupdate_scatter(rr_v, [tgt], g)
                plsc.addupdate_scatter(dr_v, [tgt], me)
            plsc.parallel_loop(0, NCHUNK, unroll=4)(p1)

            def p2(i):
                s = pl.ds(i * LN, LN)
                er = jnp.exp(jnp.abs(rr_v[s]))
                er_v[s] = er
                plsc.addupdate_scatter(denr_v, [sid_v[s]], er)
            plsc.parallel_loop(0, NCHUNK, unroll=4)(p2)

            def p3(i):
                s = pl.ds(i * LN, LN)
                sb = sid_v[s]
                r_v[s] = c100 * er_v[s] / (plsc.load_gather(denr_v, [sb])
                                           + 1e-10)
                a_v[s] = c100 * ea_v[s] / plsc.load_gather(dena_v, [sb])
                d_v[s] = jnp.where(dr_v[s] > 0.0, 1.0, 0.0).astype(f32)
                rr_v[s] = zero
                dr_v[s] = zero
            plsc.parallel_loop(0, NCHUNK, unroll=4)(p3)

            def p4(i, a):
                s = pl.ds(i * LN, LN)
                pij = pij_v[s]
                vi = pij & 0xFFFF
                vj = pij >> 16
                drr = plsc.load_gather(r_v, [vj]) - plsc.load_gather(r_v, [vi])
                daa = plsc.load_gather(a_v, [vj]) - plsc.load_gather(a_v, [vi])
                qr = jnp.exp(jnp.minimum(drr, 30.0))
                qa = jnp.exp(jnp.minimum(daa, 30.0))
                m = plsc.load_gather(d_v, [vi]) * plsc.load_gather(d_v, [vj])
                v = (qr - qa) / ((qr + 1.0) * (qa + 1.0)) * m
                return a + v * v
            acc = plsc.parallel_loop(0, NPV, unroll=8, carry=acc)(p4)

        acc_v[...] = acc
        pltpu.sync_copy(acc_v, out_h.at[wid])

    return k(slab, w, sid, tbl, pij)


def kernel(sentences, scalars_flat, rules_flat, mask_flat, indices_flat,
           mat_weight):
    perm = jnp.asarray(_ST["perm"])
    md = jnp.asarray(_ST["md"])
    scp = jnp.take(scalars_flat.astype(jnp.float32), perm, axis=1)
    mkp = jnp.take(mask_flat.astype(jnp.float32), perm, axis=1) * md[None, :]
    rup = jnp.take(rules_flat.astype(jnp.int32), perm, axis=1)
    ixp = jnp.take(indices_flat.astype(jnp.int32), perm, axis=1)
    slab = jnp.concatenate(
        [scp, mkp,
         jax.lax.bitcast_convert_type(rup, jnp.float32),
         jax.lax.bitcast_convert_type(ixp, jnp.float32),
         jnp.zeros((scp.shape[0], 64), jnp.float32)], axis=1)
    w = jnp.pad(mat_weight[:, 0].astype(jnp.float32),
                (0, 2504 - mat_weight.shape[0]))
    out = _sc_loss(slab, w, jnp.asarray(_ST["sid"]), jnp.asarray(_ST["tbl"]),
                   jnp.asarray(_ST["pij"]))
    return 0.04 * jnp.sum(out)


# speedup vs baseline: 1.6493x; 1.6493x over previous
import functools

import numpy as np
import jax
import jax.numpy as jnp
from jax import lax
from jax.experimental import pallas as pl
from jax.experimental.pallas import tpu as pltpu
from jax.experimental.pallas import tpu_sc as plsc

L = 24
NC, NS, LN = 2, 16, 16
NW = NC * NS
WMAX = L - 1


def _build_static():
    widths = []
    for level in range(1, L):
        for _pos in range(L - level):
            widths.append(level)
    widths = np.asarray(widths, np.int64)
    n_seg = len(widths)
    offs = np.concatenate([[0], np.cumsum(widths)[:-1]])

    perm, seg_id = [], []
    ppos = {}
    for p in range(int(widths.max())):
        for s in np.nonzero(widths > p)[0]:
            ppos[(s, p)] = len(perm)
            perm.append(offs[s] + p)
            seg_id.append(s)
        while len(perm) % LN:
            t = len(perm)
            perm.append(2300)
            seg_id.append(n_seg + (t % LN))
    t2 = len(perm)
    sid = np.asarray(seg_id, np.int32)
    for c in range(t2 // LN):
        assert len(set(sid[c * LN:(c + 1) * LN].tolist())) == LN

    ntbl = (n_seg + LN) * WMAX
    ntbl += (LN - ntbl % LN) % LN
    tbl = np.zeros((ntbl,), np.int32)
    for (s, p), v in ppos.items():
        tbl[s * WMAX + p] = v
    for k in range(LN):
        tbl[(n_seg + k) * WMAX] = t2 + k

    pi, pj = [], []
    for s in range(n_seg):
        for i in range(int(widths[s])):
            for j in range(i + 1, int(widths[s])):
                pi.append(ppos[(s, i)])
                pj.append(ppos[(s, j)])
    while len(pi) % LN:
        pi.append(0)
        pj.append(0)
    pij = (np.asarray(pi, np.int64)
           | (np.asarray(pj, np.int64) << 16)).astype(np.int32)
    nbin = -(-(n_seg + LN) // LN) * LN
    return dict(
        t2=t2, nbin=nbin, perm=np.asarray(perm, np.int32),
        sid=sid, tbl=tbl, pij=pij)


_ST = _build_static()
T2 = _ST["t2"]
NPV = len(_ST["pij"]) // LN
NBIN = _ST["nbin"]
NCHUNK = T2 // LN
NTBL = len(_ST["tbl"])
RRN = T2 + LN
TPAD = 2304
SLAB = 4 * TPAD


def _sc_loss(slab, w, perm, sid, tbl, pij):
    b = slab.shape[0]
    rows_per = b // NW
    mesh = plsc.VectorSubcoreMesh(core_axis_name="c", subcore_axis_name="s")
    f32 = jnp.float32
    i32 = jnp.int32
    c100 = f32(100.0)

    @functools.partial(
        pl.kernel,
        out_type=jax.ShapeDtypeStruct((NW, LN), f32),
        mesh=mesh,
        compiler_params=pltpu.CompilerParams(needs_layout_passes=False),
        scratch_types=[
            pltpu.VMEM((w.shape[0],), f32),
            pltpu.VMEM((T2,), i32),
            pltpu.VMEM((T2,), i32),
            pltpu.VMEM((NTBL,), i32),
            pltpu.VMEM((len(_ST["pij"]),), i32),
            pltpu.VMEM((rows_per * SLAB,), f32),
            pltpu.VMEM((T2,), f32),
            pltpu.VMEM((T2,), f32),
            pltpu.VMEM((RRN,), f32),
            pltpu.VMEM((RRN,), f32),
            pltpu.VMEM((T2,), f32),
            pltpu.VMEM((T2,), f32),
            pltpu.VMEM((T2,), f32),
            pltpu.VMEM((NBIN,), f32),
            pltpu.VMEM((NBIN,), f32),
            pltpu.VMEM((LN,), f32),
        ] + [pltpu.SemaphoreType.DMA] * rows_per,
    )
    def k(slab_h, w_h, perm_h, sid_h, tbl_h, pij_h, out_h,
          w_v, perm_v, sid_v, tbl_v, pij_v, rows_v, ea_v, er_v, rr_v, dr_v,
          r_v, a_v, d_v, dena_v, denr_v, acc_v, *sems):
        wid = lax.axis_index("s") * NC + lax.axis_index("c")
        base = wid * rows_per
        copies = [
            pltpu.async_copy(slab_h.at[base + r],
                             rows_v.at[pl.ds(r * SLAB, SLAB)], sems[r])
            for r in range(rows_per)
        ]
        pltpu.sync_copy(w_h, w_v)
        pltpu.sync_copy(perm_h, perm_v)
        pltpu.sync_copy(sid_h, sid_v)
        pltpu.sync_copy(tbl_h, tbl_v)
        pltpu.sync_copy(pij_h, pij_v)
        zero = jnp.zeros((LN,), f32)

        def zero_big(i):
            rr_v[pl.ds(i * LN, LN)] = zero
            dr_v[pl.ds(i * LN, LN)] = zero
        plsc.parallel_loop(0, RRN // LN, unroll=4)(zero_big)

        acc = zero
        for r in range(rows_per):
            copies[r].wait()
            roff = r * SLAB

            def zero_bins(i):
                dena_v[pl.ds(i * LN, LN)] = zero
                denr_v[pl.ds(i * LN, LN)] = zero
            plsc.parallel_loop(0, NBIN // LN, unroll=4)(zero_bins)

            def p1(i):
                s = pl.ds(i * LN, LN)
                pm = perm_v[s] + roff
                ea = jnp.exp(plsc.load_gather(rows_v, [pm]))
                ea_v[s] = ea
                sb = sid_v[s]
                plsc.addupdate_scatter(dena_v, [sb], ea)
                me = plsc.load_gather(rows_v, [pm + TPAD])
                ru = plsc.bitcast(
                    plsc.load_gather(rows_v, [pm + 2 * TPAD]), i32)
                ix = plsc.bitcast(
                    plsc.load_gather(rows_v, [pm + 3 * TPAD]), i32)
                g = plsc.load_gather(w_v, [ru]) * me
                tgt = plsc.load_gather(tbl_v, [sb * WMAX + ix])
                plsc.addupdate_scatter(rr_v, [tgt], g)
                plsc.addupdate_scatter(dr_v, [tgt], me)
            plsc.parallel_loop(0, NCHUNK, unroll=4)(p1)

            def p2(i):
                s = pl.ds(i * LN, LN)
                er = jnp.exp(jnp.abs(rr_v[s]))
                er_v[s] = er
                plsc.addupdate_scatter(denr_v, [sid_v[s]], er)
            plsc.parallel_loop(0, NCHUNK, unroll=4)(p2)

            def p3(i):
                s = pl.ds(i * LN, LN)
                sb = sid_v[s]
                r_v[s] = c100 * er_v[s] / (plsc.load_gather(denr_v, [sb])
                                           + 1e-10)
                a_v[s] = c100 * ea_v[s] / plsc.load_gather(dena_v, [sb])
                d_v[s] = jnp.where(dr_v[s] > 0.0, 1.0, 0.0).astype(f32)
                rr_v[s] = zero
                dr_v[s] = zero
            plsc.parallel_loop(0, NCHUNK, unroll=4)(p3)

            def p4(i, a):
                s = pl.ds(i * LN, LN)
                pij = pij_v[s]
                vi = pij & 0xFFFF
                vj = pij >> 16
                drr = plsc.load_gather(r_v, [vj]) - plsc.load_gather(r_v, [vi])
                daa = plsc.load_gather(a_v, [vj]) - plsc.load_gather(a_v, [vi])
                qr = jnp.exp(jnp.minimum(drr, 30.0))
                qa = jnp.exp(jnp.minimum(daa, 30.0))
                m = plsc.load_gather(d_v, [vi]) * plsc.load_gather(d_v, [vj])
                v = (qr - qa) / ((qr + 1.0) * (qa + 1.0)) * m
                return a + v * v
            acc = plsc.parallel_loop(0, NPV, unroll=8, carry=acc)(p4)

        acc_v[...] = acc
        pltpu.sync_copy(acc_v, out_h.at[wid])

    return k(slab, w, perm, sid, tbl, pij)


def kernel(sentences, scalars_flat, rules_flat, mask_flat, indices_flat,
           mat_weight):
    b, t = scalars_flat.shape
    pad = ((0, 0), (0, TPAD - t))
    slab = jnp.concatenate(
        [jnp.pad(scalars_flat.astype(jnp.float32), pad),
         jnp.pad(mask_flat.astype(jnp.float32), pad),
         jax.lax.bitcast_convert_type(
             jnp.pad(rules_flat.astype(jnp.int32), pad), jnp.float32),
         jax.lax.bitcast_convert_type(
             jnp.pad(indices_flat.astype(jnp.int32), pad), jnp.float32)],
        axis=1)
    w = jnp.pad(mat_weight[:, 0].astype(jnp.float32),
                (0, 2504 - mat_weight.shape[0]))
    out = _sc_loss(slab, w, jnp.asarray(_ST["perm"]), jnp.asarray(_ST["sid"]),
                   jnp.asarray(_ST["tbl"]), jnp.asarray(_ST["pij"]))
    return 0.04 * jnp.sum(out)


# speedup vs baseline: 1.6588x; 1.0057x over previous
import functools

import numpy as np
import jax
import jax.numpy as jnp
from jax import lax
from jax.experimental import pallas as pl
from jax.experimental.pallas import tpu as pltpu
from jax.experimental.pallas import tpu_sc as plsc

L = 24
NC, NS, LN = 2, 16, 16
NW = NC * NS
WMAX = L - 1


def _build_static():
    widths = []
    for level in range(1, L):
        for _pos in range(L - level):
            widths.append(level)
    widths = np.asarray(widths, np.int64)
    n_seg = len(widths)
    offs = np.concatenate([[0], np.cumsum(widths)[:-1]])

    perm, seg_id = [], []
    ppos = {}
    for p in range(int(widths.max())):
        for s in np.nonzero(widths > p)[0]:
            ppos[(s, p)] = len(perm)
            perm.append(offs[s] + p)
            seg_id.append(s)
        while len(perm) % LN:
            t = len(perm)
            perm.append(2300)
            seg_id.append(n_seg + (t % LN))
    t2 = len(perm)
    sid = np.asarray(seg_id, np.int32)
    for c in range(t2 // LN):
        assert len(set(sid[c * LN:(c + 1) * LN].tolist())) == LN

    ntbl = (n_seg + LN) * WMAX
    ntbl += (LN - ntbl % LN) % LN
    tbl = np.zeros((ntbl,), np.int32)
    for (s, p), v in ppos.items():
        tbl[s * WMAX + p] = v
    for k in range(LN):
        tbl[(n_seg + k) * WMAX] = t2 + k

    pi, pj = [], []
    for s in range(n_seg):
        for i in range(int(widths[s])):
            for j in range(i + 1, int(widths[s])):
                pi.append(ppos[(s, i)])
                pj.append(ppos[(s, j)])
    while len(pi) % LN:
        pi.append(0)
        pj.append(0)
    pij = (np.asarray(pi, np.int64)
           | (np.asarray(pj, np.int64) << 16)).astype(np.int32)
    nbin = -(-(n_seg + LN) // LN) * LN
    return dict(
        t2=t2, nbin=nbin, perm=np.asarray(perm, np.int32),
        sid=sid, tbl=tbl, pij=pij)


_ST = _build_static()
T2 = _ST["t2"]
NPV = len(_ST["pij"]) // LN
NBIN = _ST["nbin"]
NCHUNK = T2 // LN
NTBL = len(_ST["tbl"])
RRN = T2 + LN
TPAD = 2304
SLAB = 4 * TPAD


def _sc_loss(slab_f, slab_i, w, perm, sid, tbl, pij):
    b = slab_f.shape[0]
    rows_per = b // NW
    mesh = plsc.VectorSubcoreMesh(core_axis_name="c", subcore_axis_name="s")
    f32 = jnp.float32
    i32 = jnp.int32
    c100 = f32(100.0)

    @functools.partial(
        pl.kernel,
        out_type=jax.ShapeDtypeStruct((NW, LN), f32),
        mesh=mesh,
        compiler_params=pltpu.CompilerParams(needs_layout_passes=False),
        scratch_types=[
            pltpu.VMEM((w.shape[0],), f32),
            pltpu.VMEM((T2,), i32),
            pltpu.VMEM((T2,), i32),
            pltpu.VMEM((NTBL,), i32),
            pltpu.VMEM((len(_ST["pij"]),), i32),
        ] + [pltpu.VMEM((SLAB // 2,), f32)] * 8
          + [pltpu.VMEM((SLAB // 2,), i32)] * 8 + [
            pltpu.VMEM((T2,), f32),
            pltpu.VMEM((T2,), f32),
            pltpu.VMEM((RRN,), f32),
            pltpu.VMEM((RRN,), f32),
            pltpu.VMEM((T2,), f32),
            pltpu.VMEM((T2,), f32),
            pltpu.VMEM((T2,), f32),
            pltpu.VMEM((NBIN,), f32),
            pltpu.VMEM((NBIN,), f32),
            pltpu.VMEM((LN,), f32),
        ] + [pltpu.SemaphoreType.DMA] * (2 * rows_per),
    )
    def k(slabf_h, slabi_h, w_h, perm_h, sid_h, tbl_h, pij_h, out_h,
          w_v, perm_v, sid_v, tbl_v, pij_v, *rest):
        frows = rest[:8]
        irows = rest[8:16]
        (ea_v, er_v, rr_v, dr_v, r_v, a_v, d_v, dena_v, denr_v,
         acc_v) = rest[16:26]
        sems = rest[26:]
        wid = lax.axis_index("s") * NC + lax.axis_index("c")
        base = wid * rows_per
        copies = []
        for r in range(rows_per):
            copies.append(
                pltpu.async_copy(slabf_h.at[base + r], frows[r], sems[2 * r]))
            copies.append(
                pltpu.async_copy(slabi_h.at[base + r], irows[r],
                                 sems[2 * r + 1]))
        pltpu.sync_copy(w_h, w_v)
        pltpu.sync_copy(perm_h, perm_v)
        pltpu.sync_copy(sid_h, sid_v)
        pltpu.sync_copy(tbl_h, tbl_v)
        pltpu.sync_copy(pij_h, pij_v)
        zero = jnp.zeros((LN,), f32)

        def zero_big(i):
            rr_v[pl.ds(i * LN, LN)] = zero
            dr_v[pl.ds(i * LN, LN)] = zero
        plsc.parallel_loop(0, RRN // LN, unroll=4)(zero_big)

        acc = zero
        for r in range(rows_per):
            copies[2 * r].wait()
            copies[2 * r + 1].wait()
            ref_a = frows[r]
            ref_b = irows[r]

            def zero_bins(i):
                dena_v[pl.ds(i * LN, LN)] = zero
                denr_v[pl.ds(i * LN, LN)] = zero
            plsc.parallel_loop(0, NBIN // LN, unroll=4)(zero_bins)

            def p1(i):
                s = pl.ds(i * LN, LN)
                pm = perm_v[s]
                ea = jnp.exp(plsc.load_gather(ref_a, [pm]))
                ea_v[s] = ea
                sb = sid_v[s]
                plsc.addupdate_scatter(dena_v, [sb], ea)
                me = plsc.load_gather(ref_a, [pm + TPAD])
                ru = plsc.load_gather(ref_b, [pm])
                ix = plsc.load_gather(ref_b, [pm + TPAD])
                g = plsc.load_gather(w_v, [ru]) * me
                tgt = plsc.load_gather(tbl_v, [sb * WMAX + ix])
                plsc.addupdate_scatter(rr_v, [tgt], g)
                plsc.addupdate_scatter(dr_v, [tgt], me)
            plsc.parallel_loop(0, NCHUNK, unroll=4)(p1)

            def p2(i):
                s = pl.ds(i * LN, LN)
                er = jnp.exp(jnp.abs(rr_v[s]))
                er_v[s] = er
                plsc.addupdate_scatter(denr_v, [sid_v[s]], er)
            plsc.parallel_loop(0, NCHUNK, unroll=4)(p2)

            def p3(i):
                s = pl.ds(i * LN, LN)
                sb = sid_v[s]
                r_v[s] = c100 * er_v[s] / (plsc.load_gather(denr_v, [sb])
                                           + 1e-10)
                a_v[s] = c100 * ea_v[s] / plsc.load_gather(dena_v, [sb])
                d_v[s] = jnp.where(dr_v[s] > 0.0, 1.0, 0.0).astype(f32)
                rr_v[s] = zero
                dr_v[s] = zero
            plsc.parallel_loop(0, NCHUNK, unroll=4)(p3)

            def p4(i, a):
                s = pl.ds(i * LN, LN)
                pij = pij_v[s]
                vi = pij & 0xFFFF
                vj = pij >> 16
                drr = plsc.load_gather(r_v, [vj]) - plsc.load_gather(r_v, [vi])
                daa = plsc.load_gather(a_v, [vj]) - plsc.load_gather(a_v, [vi])
                qr = jnp.exp(jnp.minimum(drr, 30.0))
                qa = jnp.exp(jnp.minimum(daa, 30.0))
                m = plsc.load_gather(d_v, [vi]) * plsc.load_gather(d_v, [vj])
                v = (qr - qa) / ((qr + 1.0) * (qa + 1.0)) * m
                return a + v * v
            acc = plsc.parallel_loop(0, NPV, unroll=8, carry=acc)(p4)

        acc_v[...] = acc
        pltpu.sync_copy(acc_v, out_h.at[wid])

    return k(slab_f, slab_i, w, perm, sid, tbl, pij)


def kernel(sentences, scalars_flat, rules_flat, mask_flat, indices_flat,
           mat_weight):
    b, t = scalars_flat.shape
    pad = ((0, 0), (0, TPAD - t))
    slab_f = jnp.concatenate(
        [jnp.pad(scalars_flat.astype(jnp.float32), pad),
         jnp.pad(mask_flat.astype(jnp.float32), pad)], axis=1)
    slab_i = jnp.concatenate(
        [jnp.pad(rules_flat.astype(jnp.int32), pad),
         jnp.pad(indices_flat.astype(jnp.int32), pad)], axis=1)
    w = jnp.pad(mat_weight[:, 0].astype(jnp.float32),
                (0, 2504 - mat_weight.shape[0]))
    out = _sc_loss(slab_f, slab_i, w, jnp.asarray(_ST["perm"]), jnp.asarray(_ST["sid"]),
                   jnp.asarray(_ST["tbl"]), jnp.asarray(_ST["pij"]))
    return 0.04 * jnp.sum(out)
